# baseline (speedup 1.0000x reference)
import functools

import jax
import jax.numpy as jnp
from jax import lax
from jax.experimental import pallas as pl
from jax.experimental.pallas import tpu as pltpu
from jax.experimental.pallas import tpu_sc as plsc

DIM = 256
L1W = DIM - 4
NB, SEQ = 4096, 50
B = NB * SEQ
NC, NS = 2, 16
NW = NC * NS
BPW = B // NW
CH = 128
NCHUNK = BPW // CH
NBUF = 3
DEPTH = NBUF - 1


def _prep_body(l1s_ref, vc_ref, l1t_ref, vt_ref, fused_ref, comb_ref):
    l1 = jnp.clip(l1s_ref[...].astype(jnp.int32), 0, 255)
    v = jnp.clip(vc_ref[...].astype(jnp.int32), 0, 3)
    fused_ref[...] = l1 * 4 + v
    t = l1t_ref[...]
    comb_ref[:, :L1W] = jnp.broadcast_to(t[:, None, :], (256, 4, L1W)).reshape(1024, L1W)
    vt = vt_ref[...]
    comb_ref[:, L1W:] = jnp.broadcast_to(vt[None, :, :], (256, 4, 4)).reshape(1024, 4)


_prep = pl.pallas_call(
    _prep_body,
    out_shape=[
        jax.ShapeDtypeStruct((NB, SEQ), jnp.int32),
        jax.ShapeDtypeStruct((1024, DIM), jnp.float32),
    ],
)


@functools.cache
def _make_sc_gather():
    @functools.partial(
        pl.kernel,
        out_type=jax.ShapeDtypeStruct((B, DIM), jnp.float32),
        mesh=plsc.VectorSubcoreMesh(core_axis_name="c", subcore_axis_name="s"),
        scratch_types=(
            [pltpu.VMEM((NCHUNK, CH), jnp.int32), pltpu.VMEM((NCHUNK, CH), jnp.int32)]
            + [pltpu.VMEM((CH, DIM), jnp.float32) for _ in range(NBUF)]
            + [pltpu.SemaphoreType.DMA for _ in range(2 * NBUF)]
        ),
    )
    def _sc_gather(tbl_hbm, idx_hbm, didx_hbm, out_hbm, idx_v, didx_v, *bufs):
        rows = bufs[:NBUF]
        gsems = bufs[NBUF:2 * NBUF]
        ssems = bufs[2 * NBUF:]
        wid = lax.axis_index("s") * NC + lax.axis_index("c")
        base = wid * BPW
        pltpu.sync_copy(idx_hbm.at[wid], idx_v)
        pltpu.sync_copy(didx_hbm.at[wid], didx_v)

        def gwait(b):
            pltpu.make_async_copy(tbl_hbm.at[pl.ds(0, CH)], rows[b], gsems[b]).wait()

        def swait(b):
            pltpu.make_async_copy(tbl_hbm.at[pl.ds(0, CH)], rows[b], ssems[b]).wait()

        for k in range(min(DEPTH, NCHUNK)):
            pltpu.async_copy(tbl_hbm.at[idx_v.at[k]], rows[k % NBUF], gsems[k % NBUF])

        for j in range(NCHUNK):
            b = j % NBUF
            gwait(b)
            if j % 2 == 0:
                pltpu.async_copy(rows[b], out_hbm.at[pl.ds(base + j * CH, CH)], ssems[b])
            else:
                pltpu.async_copy(rows[b], out_hbm.at[didx_v.at[j]], ssems[b])
            k = j + DEPTH
            if k < NCHUNK:
                bb = k % NBUF
                if k >= NBUF:
                    swait(bb)
                pltpu.async_copy(tbl_hbm.at[idx_v.at[k]], rows[bb], gsems[bb])

        for j in range(max(0, NCHUNK - NBUF), NCHUNK):
            swait(j % NBUF)

    return _sc_gather


def kernel(l1_states, vertex_charges, l1_table, vertex_table):
    fused, comb = _prep(
        l1_states.astype(jnp.int32),
        vertex_charges.astype(jnp.int32),
        l1_table,
        vertex_table,
    )
    idx3 = fused.reshape(NW, NCHUNK, CH)
    didx3 = jnp.arange(B, dtype=jnp.int32).reshape(NW, NCHUNK, CH)
    out = _make_sc_gather()(comb, idx3, didx3)
    return out.reshape(NB, SEQ, DIM)


# speedup vs baseline: 1.0631x; 1.0631x over previous
import functools

import jax
import jax.numpy as jnp
from jax import lax
from jax.experimental import pallas as pl
from jax.experimental.pallas import tpu as pltpu
from jax.experimental.pallas import tpu_sc as plsc

DIM = 256
L1W = DIM - 4
NB, SEQ = 4096, 50
B = NB * SEQ
NC, NS = 2, 16
NW = NC * NS
BPW = B // NW
CH = 128
NCHUNK = BPW // CH
NBUF = 3
DEPTH = NBUF - 1


def _prep_body(l1s_ref, vc_ref, l1t_ref, vt_ref, fused_ref, comb_ref):
    l1 = jnp.clip(l1s_ref[...].astype(jnp.int32), 0, 255)
    v = jnp.clip(vc_ref[...].astype(jnp.int32), 0, 3)
    fused_ref[...] = l1 * 4 + v
    t = l1t_ref[...]
    comb_ref[:, :L1W] = jnp.broadcast_to(t[:, None, :], (256, 4, L1W)).reshape(1024, L1W)
    vt = vt_ref[...]
    comb_ref[:, L1W:] = jnp.broadcast_to(vt[None, :, :], (256, 4, 4)).reshape(1024, 4)


_prep = pl.pallas_call(
    _prep_body,
    out_shape=[
        jax.ShapeDtypeStruct((NB, SEQ), jnp.int32),
        jax.ShapeDtypeStruct((1024, DIM), jnp.float32),
    ],
)


@functools.cache
def _make_sc_gather():
    @functools.partial(
        pl.kernel,
        out_type=jax.ShapeDtypeStruct((B, DIM), jnp.float32),
        mesh=plsc.VectorSubcoreMesh(core_axis_name="c", subcore_axis_name="s"),
        scratch_types=(
            [pltpu.VMEM((NCHUNK, CH), jnp.int32)]
            + [pltpu.VMEM((CH, DIM), jnp.float32) for _ in range(NBUF)]
            + [pltpu.SemaphoreType.DMA for _ in range(2 * NBUF)]
        ),
    )
    def _sc_gather(tbl_hbm, idx_hbm, out_hbm, idx_v, *bufs):
        rows = bufs[:NBUF]
        gsems = bufs[NBUF:2 * NBUF]
        ssems = bufs[2 * NBUF:]
        wid = lax.axis_index("s") * NC + lax.axis_index("c")
        base = wid * BPW
        pltpu.sync_copy(idx_hbm.at[wid], idx_v)

        def gwait(b):
            pltpu.make_async_copy(tbl_hbm.at[pl.ds(0, CH)], rows[b], gsems[b]).wait()

        def swait(b):
            pltpu.make_async_copy(tbl_hbm.at[pl.ds(0, CH)], rows[b], ssems[b]).wait()

        for k in range(min(DEPTH, NCHUNK)):
            pltpu.async_copy(tbl_hbm.at[idx_v.at[k]], rows[k % NBUF], gsems[k % NBUF])

        for j in range(NCHUNK):
            b = j % NBUF
            gwait(b)
            pltpu.async_copy(rows[b], out_hbm.at[pl.ds(base + j * CH, CH)], ssems[b])
            k = j + DEPTH
            if k < NCHUNK:
                bb = k % NBUF
                if k >= NBUF:
                    swait(bb)
                pltpu.async_copy(tbl_hbm.at[idx_v.at[k]], rows[bb], gsems[bb])

        for j in range(max(0, NCHUNK - NBUF), NCHUNK):
            swait(j % NBUF)

    return _sc_gather


def kernel(l1_states, vertex_charges, l1_table, vertex_table):
    fused, comb = _prep(
        l1_states.astype(jnp.int32),
        vertex_charges.astype(jnp.int32),
        l1_table,
        vertex_table,
    )
    idx3 = fused.reshape(NW, NCHUNK, CH)
    out = _make_sc_gather()(comb, idx3)
    return out.reshape(NB, SEQ, DIM)
